# baseline (speedup 1.0000x reference)
import functools

import jax
import jax.numpy as jnp
from jax import lax
from jax.experimental import pallas as pl
from jax.experimental.pallas import tpu as pltpu
from jax.experimental.pallas import tpu_sc as plsc

VOCAB = 100000
EMBED = 64
N, T = 4096, 50

_INFO = plsc.get_sparse_core_info()
NC, NS = _INFO.num_cores, _INFO.num_subcores
NW = NC * NS
RPW = N // NW
RPC = 16
NCHUNK = RPW // RPC

_mesh = plsc.VectorSubcoreMesh(core_axis_name="c", subcore_axis_name="s")


@functools.partial(
    pl.kernel,
    out_type=jax.ShapeDtypeStruct((N, T, EMBED), jnp.float32),
    mesh=_mesh,
    scratch_types=[
        pltpu.VMEM((RPW, T), jnp.int32),
        pltpu.VMEM((2, RPC, T, EMBED), jnp.float32),
        pltpu.SemaphoreType.DMA,
        pltpu.SemaphoreType.DMA,
        pltpu.SemaphoreType.DMA,
        pltpu.SemaphoreType.DMA,
    ],
    compiler_params=pltpu.CompilerParams(use_tc_tiling_on_sc=False),
)
def _embed_lookup(x_hbm, w_hbm, out_hbm, idx, rows, g0, g1, w0, w1):
    wid = lax.axis_index("s") * NC + lax.axis_index("c")
    base = wid * RPW

    pltpu.sync_copy(x_hbm.at[pl.ds(base, RPW)], idx)

    gsem = [g0, g1]
    wsem = [w0, w1]
    gd = [[], []]
    wd = [None, None]

    def start_gathers(c, b):
        return [
            pltpu.async_copy(
                w_hbm.at[idx.at[c * RPC + j]], rows.at[b, j], gsem[b]
            )
            for j in range(RPC)
        ]

    gd[0] = start_gathers(0, 0)
    for c in range(NCHUNK):
        b = c % 2
        for d in gd[b]:
            d.wait()
        wd[b] = pltpu.async_copy(
            rows.at[b], out_hbm.at[pl.ds(base + c * RPC, RPC)], wsem[b]
        )
        if c + 1 < NCHUNK:
            nb = (c + 1) % 2
            if wd[nb] is not None:
                wd[nb].wait()
            gd[nb] = start_gathers(c + 1, nb)
    wd[0].wait()
    wd[1].wait()


def kernel(x, W_embed):
    return _embed_lookup(x, W_embed)
